# baseline (speedup 1.0000x reference)
import functools

import jax
import jax.numpy as jnp
from jax import lax
from jax.experimental import pallas as pl
from jax.experimental.pallas import tpu as pltpu
from jax.experimental.pallas import tpu_sc as plsc

NUM_EMB = 1024
DIM = 64
BM = 512
M = 32768

NW = 32
BPW = M // NW
NCHUNK = 8
CHUNK = BPW // NCHUNK


def _dist_argmin_block(x_ref, e_ref, idx_ref):
    x = x_ref[...]
    e = e_ref[...]
    scores = jax.lax.dot_general(
        x, e, (((1,), (0,)), ((), ())), preferred_element_type=jnp.float32
    )
    esq = jnp.sum(e * e, axis=0, keepdims=True)
    d = esq - 2.0 * scores
    dmin = jnp.min(d, axis=1, keepdims=True)
    col = jax.lax.broadcasted_iota(jnp.int32, (BM, NUM_EMB), 1)
    idx = jnp.min(jnp.where(d <= dmin, col, NUM_EMB), axis=1)
    idx_ref[0, 0, :] = idx


def _tc_indices(flat, embeddings):
    idx = pl.pallas_call(
        _dist_argmin_block,
        grid=(M // BM,),
        in_specs=[
            pl.BlockSpec((BM, DIM), lambda i: (i, 0)),
            pl.BlockSpec((DIM, NUM_EMB), lambda i: (0, 0)),
        ],
        out_specs=pl.BlockSpec((1, 1, BM), lambda i: (i, 0, 0)),
        out_shape=jax.ShapeDtypeStruct((M // BM, 1, BM), jnp.int32),
        compiler_params=pltpu.CompilerParams(
            dimension_semantics=("arbitrary",),
        ),
    )(flat, embeddings)
    return idx.reshape(NW, NCHUNK, CHUNK)


def _sc_gather_body(table_hbm, idx_hbm, out_hbm, idx_v, rows_v, sem):
    wid = lax.axis_index("s") * 2 + lax.axis_index("c")
    pltpu.sync_copy(idx_hbm.at[wid], idx_v)
    for j in range(NCHUNK):
        pltpu.async_copy(
            table_hbm.at[idx_v.at[j]],
            rows_v.at[j % 2],
            sem,
        )
        pltpu.make_async_copy(
            table_hbm.at[idx_v.at[j]],
            rows_v.at[j % 2],
            sem,
        ).wait()
        pltpu.sync_copy(
            rows_v.at[j % 2],
            out_hbm.at[pl.ds(wid * BPW + j * CHUNK, CHUNK)],
        )


@functools.cache
def _sc_gather():
    return pl.kernel(
        _sc_gather_body,
        mesh=plsc.VectorSubcoreMesh(core_axis_name="c", subcore_axis_name="s"),
        out_type=jax.ShapeDtypeStruct((M, 2 * DIM), jnp.float32),
        scratch_types=[
            pltpu.VMEM((NCHUNK, CHUNK), jnp.int32),
            pltpu.VMEM((2, CHUNK, 2 * DIM), jnp.float32),
            pltpu.SemaphoreType.DMA,
        ],
    )


@jax.jit
def kernel(inputs, embeddings):
    flat = inputs.reshape(-1, inputs.shape[-1])
    idx = _tc_indices(flat, embeddings)
    table = jnp.pad(embeddings.T, ((0, 0), (0, DIM)))
    out = _sc_gather()(table, idx)
    return out[:, :DIM].reshape(inputs.shape)


# speedup vs baseline: 1.2460x; 1.2460x over previous
import functools

import jax
import jax.numpy as jnp
from jax import lax
from jax.experimental import pallas as pl
from jax.experimental.pallas import tpu as pltpu
from jax.experimental.pallas import tpu_sc as plsc

NUM_EMB = 1024
DIM = 64
BM = 512
M = 32768

NW = 32
BPW = M // NW
NCHUNK = 8
CHUNK = BPW // NCHUNK


def _dist_argmin_block(x_ref, e_ref, idx_ref, esq_ref):
    @pl.when(pl.program_id(0) == 0)
    def _():
        e = e_ref[...]
        esq_ref[...] = jnp.sum(e * e, axis=0, keepdims=True)

    scores = jax.lax.dot_general(
        x_ref[...], e_ref[...], (((1,), (0,)), ((), ())),
        preferred_element_type=jnp.float32,
    )
    v = (scores + scores) - esq_ref[...]
    idx_ref[0, 0, :] = jnp.argmax(v, axis=1).astype(jnp.int32)


def _tc_indices(flat, embeddings):
    idx = pl.pallas_call(
        _dist_argmin_block,
        grid=(M // BM,),
        in_specs=[
            pl.BlockSpec((BM, DIM), lambda i: (i, 0)),
            pl.BlockSpec((DIM, NUM_EMB), lambda i: (0, 0)),
        ],
        out_specs=pl.BlockSpec((1, 1, BM), lambda i: (i, 0, 0)),
        out_shape=jax.ShapeDtypeStruct((M // BM, 1, BM), jnp.int32),
        scratch_shapes=[pltpu.VMEM((1, NUM_EMB), jnp.float32)],
        compiler_params=pltpu.CompilerParams(
            dimension_semantics=("arbitrary",),
        ),
    )(flat, embeddings)
    return idx.reshape(NW, NCHUNK, CHUNK)


def _sc_gather_body(table_hbm, idx_hbm, out_hbm, idx_v, rows_v, sem):
    wid = lax.axis_index("s") * 2 + lax.axis_index("c")
    pltpu.sync_copy(idx_hbm.at[wid], idx_v)
    for j in range(NCHUNK):
        pltpu.async_copy(
            table_hbm.at[idx_v.at[j]],
            rows_v.at[j % 2],
            sem,
        )
        pltpu.make_async_copy(
            table_hbm.at[idx_v.at[j]],
            rows_v.at[j % 2],
            sem,
        ).wait()
        pltpu.sync_copy(
            rows_v.at[j % 2],
            out_hbm.at[pl.ds(wid * BPW + j * CHUNK, CHUNK)],
        )


@functools.cache
def _sc_gather():
    return pl.kernel(
        _sc_gather_body,
        mesh=plsc.VectorSubcoreMesh(core_axis_name="c", subcore_axis_name="s"),
        out_type=jax.ShapeDtypeStruct((M, 2 * DIM), jnp.float32),
        scratch_types=[
            pltpu.VMEM((NCHUNK, CHUNK), jnp.int32),
            pltpu.VMEM((2, CHUNK, 2 * DIM), jnp.float32),
            pltpu.SemaphoreType.DMA,
        ],
    )


@jax.jit
def kernel(inputs, embeddings):
    flat = inputs.reshape(-1, inputs.shape[-1])
    idx = _tc_indices(flat, embeddings)
    table = jnp.pad(embeddings.T, ((0, 0), (0, DIM)))
    out = _sc_gather()(table, idx)
    return out[:, :DIM].reshape(inputs.shape)


# speedup vs baseline: 1.3074x; 1.0493x over previous
import functools

import jax
import jax.numpy as jnp
from jax.experimental import pallas as pl
from jax.experimental.pallas import tpu as pltpu

NUM_EMB = 1024
DIM = 64
BM = 512
M = 32768


def _vq_block(x_ref, e_ref, et_ref, o_ref, esq_ref):
    @pl.when(pl.program_id(0) == 0)
    def _():
        e = e_ref[...]
        esq_ref[...] = jnp.sum(e * e, axis=0, keepdims=True)

    scores = jax.lax.dot_general(
        x_ref[...], e_ref[...], (((1,), (0,)), ((), ())),
        preferred_element_type=jnp.float32,
    )
    v = (scores + scores) - esq_ref[...]
    idx = jnp.argmax(v, axis=1).astype(jnp.int32)
    col = jax.lax.broadcasted_iota(jnp.int32, (BM, NUM_EMB), 1)
    onehot = (col == idx[:, None]).astype(jnp.float32)
    o_ref[...] = jax.lax.dot_general(
        onehot, et_ref[...], (((1,), (0,)), ((), ())),
        preferred_element_type=jnp.float32,
    )


@jax.jit
def kernel(inputs, embeddings):
    flat = inputs.reshape(-1, inputs.shape[-1])
    et = embeddings.T
    out = pl.pallas_call(
        _vq_block,
        grid=(M // BM,),
        in_specs=[
            pl.BlockSpec((BM, DIM), lambda i: (i, 0)),
            pl.BlockSpec((DIM, NUM_EMB), lambda i: (0, 0)),
            pl.BlockSpec((NUM_EMB, DIM), lambda i: (0, 0)),
        ],
        out_specs=pl.BlockSpec((BM, DIM), lambda i: (i, 0)),
        out_shape=jax.ShapeDtypeStruct((M, DIM), jnp.float32),
        scratch_shapes=[pltpu.VMEM((1, NUM_EMB), jnp.float32)],
        compiler_params=pltpu.CompilerParams(
            dimension_semantics=("arbitrary",),
        ),
    )(flat, embeddings, et)
    return out.reshape(inputs.shape)


# speedup vs baseline: 1.3255x; 1.0138x over previous
import functools

import jax
import jax.numpy as jnp
from jax.experimental import pallas as pl
from jax.experimental.pallas import tpu as pltpu

NUM_EMB = 1024
DIM = 64
BM = 512
M = 32768


def _vq_block(x_ref, e_ref, et_ref, o_ref, esq_ref):
    @pl.when(pl.program_id(0) == 0)
    def _():
        e = e_ref[...]
        esq_ref[...] = jnp.sum(e * e, axis=0, keepdims=True)

    scores = jax.lax.dot_general(
        x_ref[...], e_ref[...], (((1,), (0,)), ((), ())),
        preferred_element_type=jnp.float32,
    )
    v = (scores + scores) - esq_ref[...]
    m = jnp.max(v, axis=1, keepdims=True)
    col = jax.lax.broadcasted_iota(jnp.int32, (BM, NUM_EMB), 1)
    idx = jnp.min(jnp.where(v >= m, col, NUM_EMB), axis=1, keepdims=True)
    onehot = (col == idx).astype(jnp.bfloat16)
    o_ref[...] = jax.lax.dot_general(
        onehot, et_ref[...], (((1,), (0,)), ((), ())),
        preferred_element_type=jnp.float32,
    )


@jax.jit
def kernel(inputs, embeddings):
    flat = inputs.reshape(-1, inputs.shape[-1])
    et = embeddings.T.astype(jnp.bfloat16)
    out = pl.pallas_call(
        _vq_block,
        grid=(M // BM,),
        in_specs=[
            pl.BlockSpec((BM, DIM), lambda i: (i, 0)),
            pl.BlockSpec((DIM, NUM_EMB), lambda i: (0, 0)),
            pl.BlockSpec((NUM_EMB, DIM), lambda i: (0, 0)),
        ],
        out_specs=pl.BlockSpec((BM, DIM), lambda i: (i, 0)),
        out_shape=jax.ShapeDtypeStruct((M, DIM), jnp.float32),
        scratch_shapes=[pltpu.VMEM((1, NUM_EMB), jnp.float32)],
        compiler_params=pltpu.CompilerParams(
            dimension_semantics=("arbitrary",),
        ),
    )(flat, embeddings, et)
    return out.reshape(inputs.shape)


# speedup vs baseline: 1.5489x; 1.1686x over previous
import functools

import jax
import jax.numpy as jnp
from jax.experimental import pallas as pl
from jax.experimental.pallas import tpu as pltpu

NUM_EMB = 1024
DIM = 64
BM = 1024
M = 32768


def _vq_block(x_ref, e_ref, et_ref, o_ref, esq_ref):
    @pl.when(pl.program_id(0) == 0)
    def _():
        e = e_ref[...]
        esq_ref[...] = 0.5 * jnp.sum(e * e, axis=0, keepdims=True)

    scores = jax.lax.dot_general(
        x_ref[...], e_ref[...], (((1,), (0,)), ((), ())),
        preferred_element_type=jnp.float32,
    )
    v = scores - esq_ref[...]
    m = jnp.max(v, axis=1, keepdims=True)
    col = jax.lax.broadcasted_iota(jnp.int32, (BM, NUM_EMB), 1)
    idx = jnp.min(jnp.where(v >= m, col, NUM_EMB), axis=1, keepdims=True)
    onehot = (col == idx).astype(jnp.bfloat16)
    o_ref[...] = jax.lax.dot_general(
        onehot, et_ref[...], (((1,), (0,)), ((), ())),
        preferred_element_type=jnp.float32,
    )


@jax.jit
def kernel(inputs, embeddings):
    flat = inputs.reshape(-1, inputs.shape[-1])
    et = embeddings.T.astype(jnp.bfloat16)
    out = pl.pallas_call(
        _vq_block,
        grid=(M // BM,),
        in_specs=[
            pl.BlockSpec((BM, DIM), lambda i: (i, 0)),
            pl.BlockSpec((DIM, NUM_EMB), lambda i: (0, 0)),
            pl.BlockSpec((NUM_EMB, DIM), lambda i: (0, 0)),
        ],
        out_specs=pl.BlockSpec((BM, DIM), lambda i: (i, 0)),
        out_shape=jax.ShapeDtypeStruct((M, DIM), jnp.float32),
        scratch_shapes=[pltpu.VMEM((1, NUM_EMB), jnp.float32)],
        compiler_params=pltpu.CompilerParams(
            dimension_semantics=("arbitrary",),
        ),
    )(flat, embeddings, et)
    return out.reshape(inputs.shape)


# speedup vs baseline: 1.6812x; 1.0854x over previous
import functools

import jax
import jax.numpy as jnp
from jax.experimental import pallas as pl
from jax.experimental.pallas import tpu as pltpu

NUM_EMB = 1024
DIM = 64
BM = 2048
M = 32768


def _vq_block(x_ref, e_ref, et_ref, o_ref, esq_ref):
    @pl.when(pl.program_id(0) == 0)
    def _():
        e = e_ref[...]
        esq_ref[...] = 0.5 * jnp.sum(e * e, axis=0, keepdims=True)

    scores = jax.lax.dot_general(
        x_ref[...], e_ref[...], (((1,), (0,)), ((), ())),
        preferred_element_type=jnp.float32,
    )
    v = scores - esq_ref[...]
    m = jnp.max(v, axis=1, keepdims=True)
    col = jax.lax.broadcasted_iota(jnp.int32, (BM, NUM_EMB), 1)
    idx = jnp.min(jnp.where(v >= m, col, NUM_EMB), axis=1, keepdims=True)
    onehot = (col == idx).astype(jnp.bfloat16)
    o_ref[...] = jax.lax.dot_general(
        onehot, et_ref[...], (((1,), (0,)), ((), ())),
        preferred_element_type=jnp.float32,
    )


@jax.jit
def kernel(inputs, embeddings):
    flat = inputs.reshape(-1, inputs.shape[-1])
    et = embeddings.T.astype(jnp.bfloat16)
    out = pl.pallas_call(
        _vq_block,
        grid=(M // BM,),
        in_specs=[
            pl.BlockSpec((BM, DIM), lambda i: (i, 0)),
            pl.BlockSpec((DIM, NUM_EMB), lambda i: (0, 0)),
            pl.BlockSpec((NUM_EMB, DIM), lambda i: (0, 0)),
        ],
        out_specs=pl.BlockSpec((BM, DIM), lambda i: (i, 0)),
        out_shape=jax.ShapeDtypeStruct((M, DIM), jnp.float32),
        scratch_shapes=[pltpu.VMEM((1, NUM_EMB), jnp.float32)],
        compiler_params=pltpu.CompilerParams(
            dimension_semantics=("arbitrary",),
        ),
    )(flat, embeddings, et)
    return out.reshape(inputs.shape)


# speedup vs baseline: 1.7492x; 1.0404x over previous
import functools

import jax
import jax.numpy as jnp
from jax.experimental import pallas as pl
from jax.experimental.pallas import tpu as pltpu

NUM_EMB = 1024
DIM = 64
BM = 4096
M = 32768


def _vq_block(x_ref, e_ref, et_ref, o_ref, esq_ref):
    @pl.when(pl.program_id(0) == 0)
    def _():
        e = e_ref[...]
        esq_ref[...] = 0.5 * jnp.sum(e * e, axis=0, keepdims=True)

    scores = jax.lax.dot_general(
        x_ref[...], e_ref[...], (((1,), (0,)), ((), ())),
        preferred_element_type=jnp.float32,
    )
    v = scores - esq_ref[...]
    m = jnp.max(v, axis=1, keepdims=True)
    col = jax.lax.broadcasted_iota(jnp.int32, (BM, NUM_EMB), 1)
    idx = jnp.min(jnp.where(v >= m, col, NUM_EMB), axis=1, keepdims=True)
    onehot = (col == idx).astype(jnp.bfloat16)
    o_ref[...] = jax.lax.dot_general(
        onehot, et_ref[...], (((1,), (0,)), ((), ())),
        preferred_element_type=jnp.float32,
    )


@jax.jit
def kernel(inputs, embeddings):
    flat = inputs.reshape(-1, inputs.shape[-1])
    et = embeddings.T.astype(jnp.bfloat16)
    out = pl.pallas_call(
        _vq_block,
        grid=(M // BM,),
        in_specs=[
            pl.BlockSpec((BM, DIM), lambda i: (i, 0)),
            pl.BlockSpec((DIM, NUM_EMB), lambda i: (0, 0)),
            pl.BlockSpec((NUM_EMB, DIM), lambda i: (0, 0)),
        ],
        out_specs=pl.BlockSpec((BM, DIM), lambda i: (i, 0)),
        out_shape=jax.ShapeDtypeStruct((M, DIM), jnp.float32),
        scratch_shapes=[pltpu.VMEM((1, NUM_EMB), jnp.float32)],
        compiler_params=pltpu.CompilerParams(
            dimension_semantics=("arbitrary",),
        ),
    )(flat, embeddings, et)
    return out.reshape(inputs.shape)


# speedup vs baseline: 1.7585x; 1.0053x over previous
import functools

import jax
import jax.numpy as jnp
from jax.experimental import pallas as pl
from jax.experimental.pallas import tpu as pltpu

NUM_EMB = 1024
DIM = 64
BM = 8192
M = 32768


def _vq_block(x_ref, e_ref, et_ref, o_ref, esq_ref):
    @pl.when(pl.program_id(0) == 0)
    def _():
        e = e_ref[...]
        esq_ref[...] = 0.5 * jnp.sum(e * e, axis=0, keepdims=True)

    scores = jax.lax.dot_general(
        x_ref[...], e_ref[...], (((1,), (0,)), ((), ())),
        preferred_element_type=jnp.float32,
    )
    v = scores - esq_ref[...]
    m = jnp.max(v, axis=1, keepdims=True)
    col = jax.lax.broadcasted_iota(jnp.int32, (BM, NUM_EMB), 1)
    idx = jnp.min(jnp.where(v >= m, col, NUM_EMB), axis=1, keepdims=True)
    onehot = (col == idx).astype(jnp.bfloat16)
    o_ref[...] = jax.lax.dot_general(
        onehot, et_ref[...], (((1,), (0,)), ((), ())),
        preferred_element_type=jnp.float32,
    )


@jax.jit
def kernel(inputs, embeddings):
    flat = inputs.reshape(-1, inputs.shape[-1])
    et = embeddings.T.astype(jnp.bfloat16)
    out = pl.pallas_call(
        _vq_block,
        grid=(M // BM,),
        in_specs=[
            pl.BlockSpec((BM, DIM), lambda i: (i, 0)),
            pl.BlockSpec((DIM, NUM_EMB), lambda i: (0, 0)),
            pl.BlockSpec((NUM_EMB, DIM), lambda i: (0, 0)),
        ],
        out_specs=pl.BlockSpec((BM, DIM), lambda i: (i, 0)),
        out_shape=jax.ShapeDtypeStruct((M, DIM), jnp.float32),
        scratch_shapes=[pltpu.VMEM((1, NUM_EMB), jnp.float32)],
        compiler_params=pltpu.CompilerParams(
            dimension_semantics=("arbitrary",),
        ),
    )(flat, embeddings, et)
    return out.reshape(inputs.shape)


# speedup vs baseline: 1.9748x; 1.1230x over previous
import functools

import jax
import jax.numpy as jnp
from jax.experimental import pallas as pl
from jax.experimental.pallas import tpu as pltpu

NUM_EMB = 1024
DIM = 64
BM = 8192
M = 32768


def _vq_block(x_ref, e_ref, et_ref, o_ref, esq_ref):
    @pl.when(pl.program_id(0) == 0)
    def _():
        e = e_ref[...]
        esq_ref[...] = 0.5 * jnp.sum(e * e, axis=0, keepdims=True)

    scores = jax.lax.dot_general(
        x_ref[...], e_ref[...], (((1,), (0,)), ((), ())),
        preferred_element_type=jnp.float32,
    )
    v = scores - esq_ref[...]
    m = jnp.max(v, axis=1, keepdims=True)
    col = jax.lax.broadcasted_iota(jnp.int32, (BM, NUM_EMB), 1).astype(jnp.float32)
    idx = jnp.min(jnp.where(v >= m, col, float(NUM_EMB)), axis=1, keepdims=True)
    onehot = (col == idx).astype(jnp.bfloat16)
    o_ref[...] = jax.lax.dot_general(
        onehot, et_ref[...], (((1,), (0,)), ((), ())),
        preferred_element_type=jnp.float32,
    )


@jax.jit
def kernel(inputs, embeddings):
    flat = inputs.reshape(-1, inputs.shape[-1])
    et = embeddings.T.astype(jnp.bfloat16)
    out = pl.pallas_call(
        _vq_block,
        grid=(M // BM,),
        in_specs=[
            pl.BlockSpec((BM, DIM), lambda i: (i, 0)),
            pl.BlockSpec((DIM, NUM_EMB), lambda i: (0, 0)),
            pl.BlockSpec((NUM_EMB, DIM), lambda i: (0, 0)),
        ],
        out_specs=pl.BlockSpec((BM, DIM), lambda i: (i, 0)),
        out_shape=jax.ShapeDtypeStruct((M, DIM), jnp.float32),
        scratch_shapes=[pltpu.VMEM((1, NUM_EMB), jnp.float32)],
        compiler_params=pltpu.CompilerParams(
            dimension_semantics=("arbitrary",),
        ),
    )(flat, embeddings, et)
    return out.reshape(inputs.shape)


# speedup vs baseline: 2.1645x; 1.0961x over previous
import functools

import jax
import jax.numpy as jnp
from jax.experimental import pallas as pl
from jax.experimental.pallas import tpu as pltpu

NUM_EMB = 1024
DIM = 64
KAUG = 8
BM = 8192
M = 32768


def _vq_block(x_ref, e_ref, et_ref, o_ref, eaug_ref):
    @pl.when(pl.program_id(0) == 0)
    def _():
        e = e_ref[...]
        esq = jnp.sum(e * e, axis=0, keepdims=True)
        eaug_ref[:DIM, :] = e
        eaug_ref[DIM:, :] = jnp.broadcast_to(
            (-0.5 / KAUG) * esq, (KAUG, NUM_EMB)
        )

    ones = jnp.ones((BM, KAUG), dtype=jnp.float32)
    xaug = jnp.concatenate([x_ref[...], ones], axis=1)
    v = jax.lax.dot_general(
        xaug, eaug_ref[...], (((1,), (0,)), ((), ())),
        preferred_element_type=jnp.float32,
    )
    m = jnp.max(v, axis=1, keepdims=True)
    col = jax.lax.broadcasted_iota(jnp.int32, (BM, NUM_EMB), 1).astype(jnp.float32)
    idx = jnp.min(jnp.where(v >= m, col, float(NUM_EMB)), axis=1, keepdims=True)
    onehot = (col == idx).astype(jnp.bfloat16)
    o_ref[...] = jax.lax.dot_general(
        onehot, et_ref[...], (((1,), (0,)), ((), ())),
        preferred_element_type=jnp.float32,
    )


@jax.jit
def kernel(inputs, embeddings):
    flat = inputs.reshape(-1, inputs.shape[-1])
    et = embeddings.T.astype(jnp.bfloat16)
    out = pl.pallas_call(
        _vq_block,
        grid=(M // BM,),
        in_specs=[
            pl.BlockSpec((BM, DIM), lambda i: (i, 0)),
            pl.BlockSpec((DIM, NUM_EMB), lambda i: (0, 0)),
            pl.BlockSpec((NUM_EMB, DIM), lambda i: (0, 0)),
        ],
        out_specs=pl.BlockSpec((BM, DIM), lambda i: (i, 0)),
        out_shape=jax.ShapeDtypeStruct((M, DIM), jnp.float32),
        scratch_shapes=[pltpu.VMEM((DIM + KAUG, NUM_EMB), jnp.float32)],
        compiler_params=pltpu.CompilerParams(
            dimension_semantics=("arbitrary",),
        ),
    )(flat, embeddings, et)
    return out.reshape(inputs.shape)
